# baseline (speedup 1.0000x reference)
import jax
import jax.numpy as jnp
from jax import lax
from jax.experimental import pallas as pl
from jax.experimental.pallas import tpu as pltpu
from jax.experimental.pallas import tpu_sc as plsc

N = 10000
D = 128
E = 320000

NC = 2
NS = 16
L = 16
NW = NC * NS

CHUNK = 96
NWIN = 5
WCH = 21
CHUNKS_PER_TILE = NWIN * WCH
E_PAD = NW * CHUNKS_PER_TILE * CHUNK
N_ACC = 10112
ROWS_PER_TILE = N_ACC // NS

_SC_PARAMS = pltpu.CompilerParams(needs_layout_passes=False)
_MESH = dict(core_axis_name="c", subcore_axis_name="s")


def _sc_counts_body(dst_hbm, counts_hbm, dst_v, counts_v):
    c = lax.axis_index("c")
    s = lax.axis_index("s")

    zeros16 = jnp.zeros((L,), jnp.float32)

    def _zero(i, carry):
        counts_v[pl.ds(i * L, L)] = zeros16
        return carry
    lax.fori_loop(0, N_ACC // L, _zero, 0)

    ones16 = jnp.ones((L,), jnp.float32)
    gpc = CHUNK // L

    def _count(g, carry):
        idx = dst_v[g // gpc, pl.ds((g % gpc) * L, L)]
        plsc.addupdate_scatter(counts_v, [idx], ones16)
        return carry

    for w in range(NWIN):
        pltpu.sync_copy(dst_hbm.at[c, s, w], dst_v)
        lax.fori_loop(0, WCH * gpc, _count, 0)

    wid = s * NC + c
    pltpu.sync_copy(counts_v, counts_hbm.at[wid])


_sc_counts = pl.kernel(
    _sc_counts_body,
    out_type=jax.ShapeDtypeStruct((NW, N_ACC), jnp.float32),
    mesh=plsc.VectorSubcoreMesh(**_MESH),
    scratch_types=(
        pltpu.VMEM((WCH, CHUNK), jnp.int32),
        pltpu.VMEM((N_ACC,), jnp.float32),
    ),
    compiler_params=_SC_PARAMS,
    name="sage_sc_counts",
)


def _sc_aggregate_body(h_hbm, src_hbm, dst_hbm, partial_hbm,
                       acc_sh, src_v, dst_v, rows_a, rows_b, rows_c,
                       sem_a, sem_b, sem_c):
    c = lax.axis_index("c")
    s = lax.axis_index("s")

    zeros16 = jnp.zeros((L,), jnp.float32)

    def _zero_row(i, carry):
        for k in range(D // L):
            rows_a[i, pl.ds(k * L, L)] = zeros16
        return carry
    lax.fori_loop(0, CHUNK, _zero_row, 0)
    base = s * ROWS_PER_TILE
    for t in range(ROWS_PER_TILE // CHUNK):
        pltpu.sync_copy(rows_a, acc_sh.at[pl.ds(base + t * CHUNK, CHUNK)])
    rem = ROWS_PER_TILE % CHUNK
    pltpu.sync_copy(
        rows_a.at[pl.ds(0, rem)],
        acc_sh.at[pl.ds(base + ROWS_PER_TILE - rem, rem)])

    plsc.subcore_barrier()

    def _gather(j, buf, sem):
        pltpu.async_copy(h_hbm.at[src_v.at[j]], buf, sem)

    def _gather_wait(buf, sem):
        pltpu.make_async_copy(h_hbm.at[src_v.at[0]], buf, sem).wait()

    def _scatter(j, buf):
        pltpu.sync_copy(buf, acc_sh.at[dst_v.at[j]], add=True)

    bufs = ((rows_a, sem_a), (rows_b, sem_b), (rows_c, sem_c))
    DEPTH = len(bufs)

    def _round(i, carry):
        j0 = DEPTH * i
        for k in range(DEPTH):
            pbuf, psem = bufs[(k - 1) % DEPTH]
            _gather(jnp.minimum(j0 + k + DEPTH - 1, WCH - 1), pbuf, psem)
            buf, sem = bufs[k]
            _gather_wait(buf, sem)
            _scatter(j0 + k, buf)
        return carry

    for w in range(NWIN):
        pltpu.sync_copy(src_hbm.at[c, s, w], src_v)
        pltpu.sync_copy(dst_hbm.at[c, s, w], dst_v)
        for k, (buf, sem) in enumerate(bufs[:DEPTH - 1]):
            _gather(k, buf, sem)
        lax.fori_loop(0, WCH // DEPTH, _round, 0)
        for buf, sem in bufs[:DEPTH - 1]:
            _gather_wait(buf, sem)

    plsc.subcore_barrier()
    pltpu.sync_copy(acc_sh.at[pl.ds(base, ROWS_PER_TILE)],
                    partial_hbm.at[c].at[pl.ds(base, ROWS_PER_TILE)])


_sc_aggregate = pl.kernel(
    _sc_aggregate_body,
    out_type=jax.ShapeDtypeStruct((NC, N_ACC, D), jnp.float32),
    mesh=plsc.VectorSubcoreMesh(**_MESH),
    scratch_types=(
        pltpu.VMEM_SHARED((N_ACC, D), jnp.float32),
        pltpu.VMEM((WCH, CHUNK), jnp.int32),
        pltpu.VMEM((WCH, CHUNK), jnp.int32),
        pltpu.VMEM((CHUNK, D), jnp.float32),
        pltpu.VMEM((CHUNK, D), jnp.float32),
        pltpu.VMEM((CHUNK, D), jnp.float32),
        pltpu.SemaphoreType.DMA,
        pltpu.SemaphoreType.DMA,
        pltpu.SemaphoreType.DMA,
    ),
    compiler_params=_SC_PARAMS,
    name="sage_sc_aggregate",
)


def _tc_invcnt_body(cnt_ref, o_ref):
    cnt = jnp.sum(cnt_ref[...], axis=0)
    o_ref[...] = (1.0 / jnp.maximum(cnt, 1.0)).reshape(N_ACC, 1)


def _tc_invcnt(counts_p):
    return pl.pallas_call(
        _tc_invcnt_body,
        out_shape=jax.ShapeDtypeStruct((N_ACC, 1), jnp.float32),
    )(counts_p)


def _tc_dense_body(p_ref, inv_ref, h_ref, wl_ref, wr_ref, b_ref, o_ref):
    inv = inv_ref[...].reshape(_BR)
    mean = (p_ref[0] + p_ref[1]) * inv[:, None]
    acc = jnp.dot(mean, wl_ref[...], preferred_element_type=jnp.float32)
    acc = acc + jnp.dot(h_ref[...], wr_ref[...],
                        preferred_element_type=jnp.float32)
    acc = acc + b_ref[...]
    o_ref[...] = jnp.maximum(acc, 0.0)


_BR = 2528


def _tc_dense(partial, inv_c, h, wl, wr, b):
    return pl.pallas_call(
        _tc_dense_body,
        grid=(N_ACC // _BR,),
        in_specs=[
            pl.BlockSpec((NC, _BR, D), lambda r: (0, r, 0)),
            pl.BlockSpec((_BR, 1), lambda r: (r, 0)),
            pl.BlockSpec((_BR, D), lambda r: (r, 0)),
            pl.BlockSpec((D, D), lambda r: (0, 0)),
            pl.BlockSpec((D, D), lambda r: (0, 0)),
            pl.BlockSpec((1, D), lambda r: (0, 0)),
        ],
        out_specs=pl.BlockSpec((_BR, D), lambda r: (r, 0)),
        out_shape=jax.ShapeDtypeStruct((N_ACC, D), jnp.float32),
    )(partial, inv_c, h, wl, wr, b)


def kernel(x, edge_index, W1_l, W1_r, b1, W2_l, W2_r, b2):
    src = edge_index[0].astype(jnp.int32)
    dst = edge_index[1].astype(jnp.int32)
    pad_k = jnp.arange(E_PAD - E, dtype=jnp.int32)
    src_p = jnp.concatenate(
        [src, pad_k % N]
    ).reshape(NC, NS, NWIN, WCH, CHUNK)
    dst_p = jnp.concatenate(
        [dst, N + pad_k % (N_ACC - N)]
    ).reshape(NC, NS, NWIN, WCH, CHUNK)

    x_p = jnp.zeros((N_ACC, D), x.dtype).at[:N].set(x)
    b1_ = b1.reshape(1, D)
    b2_ = b2.reshape(1, D)

    counts_p = _sc_counts(dst_p)
    inv_c = _tc_invcnt(counts_p)
    partial1 = _sc_aggregate(x_p, src_p, dst_p)
    h1 = _tc_dense(partial1, inv_c, x_p, W1_l, W1_r, b1_)

    partial2 = _sc_aggregate(h1, src_p, dst_p)
    h2 = _tc_dense(partial2, inv_c, h1, W2_l, W2_r, b2_)

    return h2[:N]


# speedup vs baseline: 1.1052x; 1.1052x over previous
import jax
import jax.numpy as jnp
from jax import lax
from jax.experimental import pallas as pl
from jax.experimental.pallas import tpu as pltpu
from jax.experimental.pallas import tpu_sc as plsc

N = 10000
D = 128
E = 320000

NC = 2
NS = 16
L = 16
NW = NC * NS

CHUNK = 80
NWIN = 2
WCH = 63
CHUNKS_PER_TILE = NWIN * WCH
E_PAD = NW * CHUNKS_PER_TILE * CHUNK
N_ACC = 10112
ROWS_PER_TILE = N_ACC // NS

_SC_PARAMS = pltpu.CompilerParams(needs_layout_passes=False)
_MESH = dict(core_axis_name="c", subcore_axis_name="s")


def _sc_counts_body(dst_hbm, counts_hbm, dst_v, counts_v):
    c = lax.axis_index("c")
    s = lax.axis_index("s")

    zeros16 = jnp.zeros((L,), jnp.float32)

    def _zero(i, carry):
        counts_v[pl.ds(i * L, L)] = zeros16
        return carry
    lax.fori_loop(0, N_ACC // L, _zero, 0)

    ones16 = jnp.ones((L,), jnp.float32)
    gpc = CHUNK // L

    def _count(g, carry):
        idx = dst_v[g // gpc, pl.ds((g % gpc) * L, L)]
        plsc.addupdate_scatter(counts_v, [idx], ones16)
        return carry

    for w in range(NWIN):
        pltpu.sync_copy(dst_hbm.at[c, s, w], dst_v)
        lax.fori_loop(0, WCH * gpc, _count, 0)

    wid = s * NC + c
    pltpu.sync_copy(counts_v, counts_hbm.at[wid])


_sc_counts = pl.kernel(
    _sc_counts_body,
    out_type=jax.ShapeDtypeStruct((NW, N_ACC), jnp.float32),
    mesh=plsc.VectorSubcoreMesh(**_MESH),
    scratch_types=(
        pltpu.VMEM((WCH, CHUNK), jnp.int32),
        pltpu.VMEM((N_ACC,), jnp.float32),
    ),
    compiler_params=_SC_PARAMS,
    name="sage_sc_counts",
)


def _sc_aggregate_body(h_hbm, src_hbm, dst_hbm, partial_hbm,
                       acc_sh, src_v, dst_v, rows_a, rows_b, rows_c,
                       sem_a, sem_b, sem_c):
    c = lax.axis_index("c")
    s = lax.axis_index("s")

    def _gather(j, buf, sem):
        pltpu.async_copy(h_hbm.at[src_v.at[j]], buf, sem)

    def _gather_wait(buf, sem):
        pltpu.make_async_copy(h_hbm.at[src_v.at[0]], buf, sem).wait()

    def _scatter(j, buf):
        pltpu.sync_copy(buf, acc_sh.at[dst_v.at[j]], add=True)

    bufs = ((rows_a, sem_a), (rows_b, sem_b), (rows_c, sem_c))
    DEPTH = len(bufs)

    pltpu.sync_copy(src_hbm.at[c, s, 0], src_v)
    pltpu.sync_copy(dst_hbm.at[c, s, 0], dst_v)
    for k, (buf, sem) in enumerate(bufs[:DEPTH - 1]):
        _gather(k, buf, sem)

    zeros16 = jnp.zeros((L,), jnp.float32)

    def _zero_row(i, carry):
        for k in range(D // L):
            rows_c[i, pl.ds(k * L, L)] = zeros16
        return carry
    lax.fori_loop(0, CHUNK, _zero_row, 0)
    base = s * ROWS_PER_TILE
    for t in range(ROWS_PER_TILE // CHUNK):
        pltpu.sync_copy(rows_c, acc_sh.at[pl.ds(base + t * CHUNK, CHUNK)])
    rem = ROWS_PER_TILE % CHUNK
    pltpu.sync_copy(
        rows_c.at[pl.ds(0, rem)],
        acc_sh.at[pl.ds(base + ROWS_PER_TILE - rem, rem)])

    plsc.subcore_barrier()

    def _round(i, carry):
        j0 = DEPTH * i
        for k in range(DEPTH):
            pbuf, psem = bufs[(k - 1) % DEPTH]
            _gather(jnp.minimum(j0 + k + DEPTH - 1, WCH - 1), pbuf, psem)
            buf, sem = bufs[k]
            _gather_wait(buf, sem)
            _scatter(j0 + k, buf)
        return carry

    for w in range(NWIN):
        if w > 0:
            pltpu.sync_copy(src_hbm.at[c, s, w], src_v)
            pltpu.sync_copy(dst_hbm.at[c, s, w], dst_v)
            for k, (buf, sem) in enumerate(bufs[:DEPTH - 1]):
                _gather(k, buf, sem)
        lax.fori_loop(0, WCH // DEPTH, _round, 0)
        for buf, sem in bufs[:DEPTH - 1]:
            _gather_wait(buf, sem)

    plsc.subcore_barrier()
    pltpu.sync_copy(acc_sh.at[pl.ds(base, ROWS_PER_TILE)],
                    partial_hbm.at[c].at[pl.ds(base, ROWS_PER_TILE)])


_sc_aggregate = pl.kernel(
    _sc_aggregate_body,
    out_type=jax.ShapeDtypeStruct((NC, N_ACC, D), jnp.float32),
    mesh=plsc.VectorSubcoreMesh(**_MESH),
    scratch_types=(
        pltpu.VMEM_SHARED((N_ACC, D), jnp.float32),
        pltpu.VMEM((WCH, CHUNK), jnp.int32),
        pltpu.VMEM((WCH, CHUNK), jnp.int32),
        pltpu.VMEM((CHUNK, D), jnp.float32),
        pltpu.VMEM((CHUNK, D), jnp.float32),
        pltpu.VMEM((CHUNK, D), jnp.float32),
        pltpu.SemaphoreType.DMA,
        pltpu.SemaphoreType.DMA,
        pltpu.SemaphoreType.DMA,
    ),
    compiler_params=_SC_PARAMS,
    name="sage_sc_aggregate",
)


def _tc_invcnt_body(cnt_ref, o_ref):
    cnt = jnp.sum(cnt_ref[...], axis=0)
    o_ref[...] = (1.0 / jnp.maximum(cnt, 1.0)).reshape(N_ACC, 1)


def _tc_invcnt(counts_p):
    return pl.pallas_call(
        _tc_invcnt_body,
        out_shape=jax.ShapeDtypeStruct((N_ACC, 1), jnp.float32),
    )(counts_p)


def _tc_dense_body(p_ref, inv_ref, h_ref, wl_ref, wr_ref, b_ref, o_ref):
    inv = inv_ref[...].reshape(_BR)
    mean = (p_ref[0] + p_ref[1]) * inv[:, None]
    acc = jnp.dot(mean, wl_ref[...], preferred_element_type=jnp.float32)
    acc = acc + jnp.dot(h_ref[...], wr_ref[...],
                        preferred_element_type=jnp.float32)
    acc = acc + b_ref[...]
    o_ref[...] = jnp.maximum(acc, 0.0)


_BR = 2528


def _tc_dense(partial, inv_c, h, wl, wr, b):
    return pl.pallas_call(
        _tc_dense_body,
        grid=(N_ACC // _BR,),
        in_specs=[
            pl.BlockSpec((NC, _BR, D), lambda r: (0, r, 0)),
            pl.BlockSpec((_BR, 1), lambda r: (r, 0)),
            pl.BlockSpec((_BR, D), lambda r: (r, 0)),
            pl.BlockSpec((D, D), lambda r: (0, 0)),
            pl.BlockSpec((D, D), lambda r: (0, 0)),
            pl.BlockSpec((1, D), lambda r: (0, 0)),
        ],
        out_specs=pl.BlockSpec((_BR, D), lambda r: (r, 0)),
        out_shape=jax.ShapeDtypeStruct((N_ACC, D), jnp.float32),
    )(partial, inv_c, h, wl, wr, b)


def kernel(x, edge_index, W1_l, W1_r, b1, W2_l, W2_r, b2):
    src = edge_index[0].astype(jnp.int32)
    dst = edge_index[1].astype(jnp.int32)
    pad_k = jnp.arange(E_PAD - E, dtype=jnp.int32)
    src_p = jnp.concatenate(
        [src, pad_k % N]
    ).reshape(NC, NS, NWIN, WCH, CHUNK)
    dst_p = jnp.concatenate(
        [dst, N + pad_k % (N_ACC - N)]
    ).reshape(NC, NS, NWIN, WCH, CHUNK)

    x_p = jnp.zeros((N_ACC, D), x.dtype).at[:N].set(x)
    b1_ = b1.reshape(1, D)
    b2_ = b2.reshape(1, D)

    counts_p = _sc_counts(dst_p)
    inv_c = _tc_invcnt(counts_p)
    partial1 = _sc_aggregate(x_p, src_p, dst_p)
    h1 = _tc_dense(partial1, inv_c, x_p, W1_l, W1_r, b1_)

    partial2 = _sc_aggregate(h1, src_p, dst_p)
    h2 = _tc_dense(partial2, inv_c, h1, W2_l, W2_r, b2_)

    return h2[:N]


# speedup vs baseline: 1.1199x; 1.0134x over previous
import jax
import jax.numpy as jnp
from jax import lax
from jax.experimental import pallas as pl
from jax.experimental.pallas import tpu as pltpu
from jax.experimental.pallas import tpu_sc as plsc

N = 10000
D = 128
E = 320000

NC = 2
NS = 16
L = 16
NW = NC * NS

CHUNK = 80
NWIN = 2
WCH = 63
CHUNKS_PER_TILE = NWIN * WCH
E_PAD = NW * CHUNKS_PER_TILE * CHUNK
N_ACC = 10112
ROWS_PER_TILE = N_ACC // NS

_SC_PARAMS = pltpu.CompilerParams(needs_layout_passes=False)
_MESH = dict(core_axis_name="c", subcore_axis_name="s")


def _sc_counts_body(dst_hbm, counts_hbm, dst_v, counts_v):
    c = lax.axis_index("c")
    s = lax.axis_index("s")

    zeros16 = jnp.zeros((L,), jnp.float32)

    def _zero(i, carry):
        counts_v[pl.ds(i * L, L)] = zeros16
        return carry
    lax.fori_loop(0, N_ACC // L, _zero, 0)

    ones16 = jnp.ones((L,), jnp.float32)
    gpc = CHUNK // L

    def _count(g, carry):
        idx = dst_v[g // gpc, pl.ds((g % gpc) * L, L)]
        plsc.addupdate_scatter(counts_v, [idx], ones16)
        return carry

    for w in range(NWIN):
        pltpu.sync_copy(dst_hbm.at[c, s, w], dst_v)
        lax.fori_loop(0, WCH * gpc, _count, 0)

    wid = s * NC + c
    pltpu.sync_copy(counts_v, counts_hbm.at[wid])


_sc_counts = pl.kernel(
    _sc_counts_body,
    out_type=jax.ShapeDtypeStruct((NW, N_ACC), jnp.float32),
    mesh=plsc.VectorSubcoreMesh(**_MESH),
    scratch_types=(
        pltpu.VMEM((WCH, CHUNK), jnp.int32),
        pltpu.VMEM((N_ACC,), jnp.float32),
    ),
    compiler_params=_SC_PARAMS,
    name="sage_sc_counts",
)


def _sc_aggregate_body(h_hbm, src_hbm, dst_hbm, partial_hbm,
                       acc_sh, src_v, dst_v, rows_a, rows_b, rows_c,
                       sem_a, sem_b, sem_c):
    c = lax.axis_index("c")
    s = lax.axis_index("s")

    def _gather(j, buf, sem):
        pltpu.async_copy(h_hbm.at[src_v.at[j]], buf, sem)

    def _gather_wait(buf, sem):
        pltpu.make_async_copy(h_hbm.at[src_v.at[0]], buf, sem).wait()

    def _scatter(j, buf):
        pltpu.sync_copy(buf, acc_sh.at[dst_v.at[j]], add=True)

    bufs = ((rows_a, sem_a), (rows_b, sem_b), (rows_c, sem_c))
    DEPTH = len(bufs)

    pltpu.sync_copy(src_hbm.at[c, s, 0], src_v)
    pltpu.sync_copy(dst_hbm.at[c, s, 0], dst_v)
    for k, (buf, sem) in enumerate(bufs[:DEPTH - 1]):
        _gather(k, buf, sem)

    zeros16 = jnp.zeros((L,), jnp.float32)

    def _zero_row(i, carry):
        for k in range(D // L):
            rows_c[i, pl.ds(k * L, L)] = zeros16
        return carry
    lax.fori_loop(0, CHUNK, _zero_row, 0)
    base = s * ROWS_PER_TILE
    for t in range(ROWS_PER_TILE // CHUNK):
        pltpu.sync_copy(rows_c, acc_sh.at[pl.ds(base + t * CHUNK, CHUNK)])
    rem = ROWS_PER_TILE % CHUNK
    pltpu.sync_copy(
        rows_c.at[pl.ds(0, rem)],
        acc_sh.at[pl.ds(base + ROWS_PER_TILE - rem, rem)])

    plsc.subcore_barrier()

    def _round(i, carry):
        j0 = DEPTH * i
        for k in range(DEPTH):
            pbuf, psem = bufs[(k - 1) % DEPTH]
            _gather(jnp.minimum(j0 + k + DEPTH - 1, WCH - 1), pbuf, psem)
            buf, sem = bufs[k]
            _gather_wait(buf, sem)
            _scatter(j0 + k, buf)
        return carry

    for w in range(NWIN):
        if w > 0:
            pltpu.sync_copy(src_hbm.at[c, s, w], src_v)
            pltpu.sync_copy(dst_hbm.at[c, s, w], dst_v)
            for k, (buf, sem) in enumerate(bufs[:DEPTH - 1]):
                _gather(k, buf, sem)
        lax.fori_loop(0, WCH // DEPTH, _round, 0)
        for buf, sem in bufs[:DEPTH - 1]:
            _gather_wait(buf, sem)

    plsc.subcore_barrier()
    pltpu.sync_copy(acc_sh.at[pl.ds(base, ROWS_PER_TILE)],
                    partial_hbm.at[c].at[pl.ds(base, ROWS_PER_TILE)])


_sc_aggregate = pl.kernel(
    _sc_aggregate_body,
    out_type=jax.ShapeDtypeStruct((NC, N_ACC, D), jnp.float32),
    mesh=plsc.VectorSubcoreMesh(**_MESH),
    scratch_types=(
        pltpu.VMEM_SHARED((N_ACC, D), jnp.float32),
        pltpu.VMEM((WCH, CHUNK), jnp.int32),
        pltpu.VMEM((WCH, CHUNK), jnp.int32),
        pltpu.VMEM((CHUNK, D), jnp.float32),
        pltpu.VMEM((CHUNK, D), jnp.float32),
        pltpu.VMEM((CHUNK, D), jnp.float32),
        pltpu.SemaphoreType.DMA,
        pltpu.SemaphoreType.DMA,
        pltpu.SemaphoreType.DMA,
    ),
    compiler_params=_SC_PARAMS,
    name="sage_sc_aggregate",
)


def _tc_invcnt_body(cnt_ref, o_ref):
    cnt = jnp.sum(cnt_ref[...], axis=0)
    o_ref[...] = (1.0 / jnp.maximum(cnt, 1.0)).reshape(N_ACC, 1)


def _tc_invcnt(counts_p):
    return pl.pallas_call(
        _tc_invcnt_body,
        out_shape=jax.ShapeDtypeStruct((N_ACC, 1), jnp.float32),
    )(counts_p)


def _tc_dense_body(p_ref, inv_ref, h_ref, wl_ref, wr_ref, b_ref, o_ref):
    inv = inv_ref[...].reshape(_BR)
    mean = (p_ref[0] + p_ref[1]) * inv[:, None]
    acc = jnp.dot(mean, wl_ref[...], preferred_element_type=jnp.float32)
    acc = acc + jnp.dot(h_ref[...], wr_ref[...],
                        preferred_element_type=jnp.float32)
    acc = acc + b_ref[...]
    o_ref[...] = jnp.maximum(acc, 0.0)


_BR = 2000


def _tc_dense(partial, inv_c, h, wl, wr, b):
    return pl.pallas_call(
        _tc_dense_body,
        grid=(N // _BR,),
        in_specs=[
            pl.BlockSpec((NC, _BR, D), lambda r: (0, r, 0)),
            pl.BlockSpec((_BR, 1), lambda r: (r, 0)),
            pl.BlockSpec((_BR, D), lambda r: (r, 0)),
            pl.BlockSpec((D, D), lambda r: (0, 0)),
            pl.BlockSpec((D, D), lambda r: (0, 0)),
            pl.BlockSpec((1, D), lambda r: (0, 0)),
        ],
        out_specs=pl.BlockSpec((_BR, D), lambda r: (r, 0)),
        out_shape=jax.ShapeDtypeStruct((N, D), jnp.float32),
    )(partial, inv_c, h, wl, wr, b)


def kernel(x, edge_index, W1_l, W1_r, b1, W2_l, W2_r, b2):
    src = edge_index[0].astype(jnp.int32)
    dst = edge_index[1].astype(jnp.int32)
    pad_k = jnp.arange(E_PAD - E, dtype=jnp.int32)
    src_p = jnp.concatenate(
        [src, pad_k % N]
    ).reshape(NC, NS, NWIN, WCH, CHUNK)
    dst_p = jnp.concatenate(
        [dst, N + pad_k % (N_ACC - N)]
    ).reshape(NC, NS, NWIN, WCH, CHUNK)

    b1_ = b1.reshape(1, D)
    b2_ = b2.reshape(1, D)

    counts_p = _sc_counts(dst_p)
    inv_c = _tc_invcnt(counts_p)
    partial1 = _sc_aggregate(x, src_p, dst_p)
    h1 = _tc_dense(partial1, inv_c, x, W1_l, W1_r, b1_)

    partial2 = _sc_aggregate(h1, src_p, dst_p)
    return _tc_dense(partial2, inv_c, h1, W2_l, W2_r, b2_)


# speedup vs baseline: 1.1312x; 1.0101x over previous
import jax
import jax.numpy as jnp
from jax import lax
from jax.experimental import pallas as pl
from jax.experimental.pallas import tpu as pltpu
from jax.experimental.pallas import tpu_sc as plsc

N = 10000
D = 128
E = 320000

NC = 2
NS = 16
L = 16
NW = NC * NS

CHUNK = 80
NWIN = 2
WCH = 63
CHUNKS_PER_TILE = NWIN * WCH
E_PAD = NW * CHUNKS_PER_TILE * CHUNK
N_ACC = 10112
ROWS_PER_TILE = N_ACC // NS

_SC_PARAMS = pltpu.CompilerParams(needs_layout_passes=False)
_MESH = dict(core_axis_name="c", subcore_axis_name="s")


def _sc_counts_body(dst_hbm, counts_hbm, dst_a, dst_b, counts_v,
                    sem_a, sem_b):
    c = lax.axis_index("c")
    s = lax.axis_index("s")

    bufs = ((dst_a, sem_a), (dst_b, sem_b))
    pltpu.async_copy(dst_hbm.at[c, s, 0], dst_a, sem_a)

    zeros16 = jnp.zeros((L,), jnp.float32)

    def _zero(i, carry):
        counts_v[pl.ds(i * L, L)] = zeros16
        return carry
    lax.fori_loop(0, N_ACC // L, _zero, 0)

    ones16 = jnp.ones((L,), jnp.float32)
    gpc = CHUNK // L

    for w in range(NWIN):
        buf, sem = bufs[w % 2]
        pltpu.make_async_copy(dst_hbm.at[c, s, 0], buf, sem).wait()
        if w + 1 < NWIN:
            nbuf, nsem = bufs[(w + 1) % 2]
            pltpu.async_copy(dst_hbm.at[c, s, w + 1], nbuf, nsem)

        def _count(g, carry):
            idx = buf[g // gpc, pl.ds((g % gpc) * L, L)]
            plsc.addupdate_scatter(counts_v, [idx], ones16)
            return carry
        lax.fori_loop(0, WCH * gpc, _count, 0)

    wid = s * NC + c
    pltpu.sync_copy(counts_v, counts_hbm.at[wid])


_sc_counts = pl.kernel(
    _sc_counts_body,
    out_type=jax.ShapeDtypeStruct((NW, N_ACC), jnp.float32),
    mesh=plsc.VectorSubcoreMesh(**_MESH),
    scratch_types=(
        pltpu.VMEM((WCH, CHUNK), jnp.int32),
        pltpu.VMEM((WCH, CHUNK), jnp.int32),
        pltpu.VMEM((N_ACC,), jnp.float32),
        pltpu.SemaphoreType.DMA,
        pltpu.SemaphoreType.DMA,
    ),
    compiler_params=_SC_PARAMS,
    name="sage_sc_counts",
)


def _sc_aggregate_body(h_hbm, src_hbm, dst_hbm, partial_hbm,
                       acc_sh, src_v, dst_v, rows_a, rows_b, rows_c,
                       sem_a, sem_b, sem_c):
    c = lax.axis_index("c")
    s = lax.axis_index("s")

    def _gather(j, buf, sem):
        pltpu.async_copy(h_hbm.at[src_v.at[j]], buf, sem)

    def _gather_wait(buf, sem):
        pltpu.make_async_copy(h_hbm.at[src_v.at[0]], buf, sem).wait()

    def _scatter(j, buf):
        pltpu.sync_copy(buf, acc_sh.at[dst_v.at[j]], add=True)

    bufs = ((rows_a, sem_a), (rows_b, sem_b), (rows_c, sem_c))
    DEPTH = len(bufs)

    pltpu.sync_copy(src_hbm.at[c, s, 0], src_v)
    pltpu.sync_copy(dst_hbm.at[c, s, 0], dst_v)
    for k, (buf, sem) in enumerate(bufs[:DEPTH - 1]):
        _gather(k, buf, sem)

    zeros16 = jnp.zeros((L,), jnp.float32)

    def _zero_row(i, carry):
        for k in range(D // L):
            rows_c[i, pl.ds(k * L, L)] = zeros16
        return carry
    lax.fori_loop(0, CHUNK, _zero_row, 0)
    base = s * ROWS_PER_TILE
    for t in range(ROWS_PER_TILE // CHUNK):
        pltpu.sync_copy(rows_c, acc_sh.at[pl.ds(base + t * CHUNK, CHUNK)])
    rem = ROWS_PER_TILE % CHUNK
    pltpu.sync_copy(
        rows_c.at[pl.ds(0, rem)],
        acc_sh.at[pl.ds(base + ROWS_PER_TILE - rem, rem)])

    plsc.subcore_barrier()

    def _round(i, carry):
        j0 = DEPTH * i
        for k in range(DEPTH):
            pbuf, psem = bufs[(k - 1) % DEPTH]
            _gather(jnp.minimum(j0 + k + DEPTH - 1, WCH - 1), pbuf, psem)
            buf, sem = bufs[k]
            _gather_wait(buf, sem)
            _scatter(j0 + k, buf)
        return carry

    for w in range(NWIN):
        if w > 0:
            pltpu.sync_copy(src_hbm.at[c, s, w], src_v)
            pltpu.sync_copy(dst_hbm.at[c, s, w], dst_v)
            for k, (buf, sem) in enumerate(bufs[:DEPTH - 1]):
                _gather(k, buf, sem)
        lax.fori_loop(0, WCH // DEPTH, _round, 0)
        for buf, sem in bufs[:DEPTH - 1]:
            _gather_wait(buf, sem)

    plsc.subcore_barrier()
    pltpu.sync_copy(acc_sh.at[pl.ds(base, ROWS_PER_TILE)],
                    partial_hbm.at[c].at[pl.ds(base, ROWS_PER_TILE)])


_sc_aggregate = pl.kernel(
    _sc_aggregate_body,
    out_type=jax.ShapeDtypeStruct((NC, N_ACC, D), jnp.float32),
    mesh=plsc.VectorSubcoreMesh(**_MESH),
    scratch_types=(
        pltpu.VMEM_SHARED((N_ACC, D), jnp.float32),
        pltpu.VMEM((WCH, CHUNK), jnp.int32),
        pltpu.VMEM((WCH, CHUNK), jnp.int32),
        pltpu.VMEM((CHUNK, D), jnp.float32),
        pltpu.VMEM((CHUNK, D), jnp.float32),
        pltpu.VMEM((CHUNK, D), jnp.float32),
        pltpu.SemaphoreType.DMA,
        pltpu.SemaphoreType.DMA,
        pltpu.SemaphoreType.DMA,
    ),
    compiler_params=_SC_PARAMS,
    name="sage_sc_aggregate",
)


def _tc_invcnt_body(cnt_ref, o_ref):
    cnt = jnp.sum(cnt_ref[...], axis=0)
    o_ref[...] = (1.0 / jnp.maximum(cnt, 1.0)).reshape(N_ACC, 1)


def _tc_invcnt(counts_p):
    return pl.pallas_call(
        _tc_invcnt_body,
        out_shape=jax.ShapeDtypeStruct((N_ACC, 1), jnp.float32),
    )(counts_p)


def _tc_dense_body(p_ref, inv_ref, h_ref, wl_ref, wr_ref, b_ref, o_ref):
    inv = inv_ref[...].reshape(_BR)
    mean = (p_ref[0] + p_ref[1]) * inv[:, None]
    acc = jnp.dot(mean, wl_ref[...], preferred_element_type=jnp.float32)
    acc = acc + jnp.dot(h_ref[...], wr_ref[...],
                        preferred_element_type=jnp.float32)
    acc = acc + b_ref[...]
    o_ref[...] = jnp.maximum(acc, 0.0)


_BR = 2000


def _tc_dense(partial, inv_c, h, wl, wr, b):
    return pl.pallas_call(
        _tc_dense_body,
        grid=(N // _BR,),
        in_specs=[
            pl.BlockSpec((NC, _BR, D), lambda r: (0, r, 0)),
            pl.BlockSpec((_BR, 1), lambda r: (r, 0)),
            pl.BlockSpec((_BR, D), lambda r: (r, 0)),
            pl.BlockSpec((D, D), lambda r: (0, 0)),
            pl.BlockSpec((D, D), lambda r: (0, 0)),
            pl.BlockSpec((1, D), lambda r: (0, 0)),
        ],
        out_specs=pl.BlockSpec((_BR, D), lambda r: (r, 0)),
        out_shape=jax.ShapeDtypeStruct((N, D), jnp.float32),
    )(partial, inv_c, h, wl, wr, b)


def kernel(x, edge_index, W1_l, W1_r, b1, W2_l, W2_r, b2):
    src = edge_index[0].astype(jnp.int32)
    dst = edge_index[1].astype(jnp.int32)
    pad_k = jnp.arange(E_PAD - E, dtype=jnp.int32)
    src_p = jnp.concatenate(
        [src, pad_k % N]
    ).reshape(NC, NS, NWIN, WCH, CHUNK)
    dst_p = jnp.concatenate(
        [dst, N + pad_k % (N_ACC - N)]
    ).reshape(NC, NS, NWIN, WCH, CHUNK)

    b1_ = b1.reshape(1, D)
    b2_ = b2.reshape(1, D)

    counts_p = _sc_counts(dst_p)
    inv_c = _tc_invcnt(counts_p)
    partial1 = _sc_aggregate(x, src_p, dst_p)
    h1 = _tc_dense(partial1, inv_c, x, W1_l, W1_r, b1_)

    partial2 = _sc_aggregate(h1, src_p, dst_p)
    return _tc_dense(partial2, inv_c, h1, W2_l, W2_r, b2_)
